# baseline (speedup 1.0000x reference)
import jax
import jax.numpy as jnp
from jax.experimental import pallas as pl
from jax.experimental.pallas import tpu as pltpu

DIM = 1024
NUM_EXPERTS = 4
TOP_K = 2
ADJ_HIDDEN = 1368
N_TOKENS = 2048
TB = 256


def _moe_body(x_ref, wg_ref, wgate_ref, w1_ref, w2_ref, out_ref):
    e = pl.program_id(1)
    xb = x_ref[...]

    logits = jax.lax.dot_general(
        xb, wg_ref[...], (((1,), (1,)), ((), ())),
        preferred_element_type=jnp.float32,
        precision=jax.lax.Precision.DEFAULT)
    m = jnp.max(logits, axis=-1, keepdims=True)
    ex = jnp.exp(logits - m)
    probs = ex / jnp.sum(ex, axis=-1, keepdims=True)

    cols = [probs[:, c] for c in range(NUM_EXPERTS)]
    sel = []
    for c in range(NUM_EXPERTS):
        rank = jnp.zeros_like(cols[c])
        for j in range(NUM_EXPERTS):
            if j == c:
                continue
            beats = (cols[j] > cols[c]) | ((cols[j] == cols[c]) & (j < c))
            rank = rank + beats.astype(jnp.float32)
        sel.append((rank < TOP_K).astype(jnp.float32))
    sum_sel = sum(s * p for s, p in zip(sel, cols))
    tw = jnp.zeros_like(cols[0])
    for c in range(NUM_EXPERTS):
        tw = jnp.where(e == c, sel[c] * cols[c] / (sum_sel + 1e-8), tw)

    wgate = wgate_ref[0]
    w1 = w1_ref[0]
    w2 = w2_ref[0]
    g = jax.lax.dot_general(
        xb, wgate, (((1,), (1,)), ((), ())),
        preferred_element_type=jnp.float32,
        precision=jax.lax.Precision.HIGHEST)
    u = jax.lax.dot_general(
        xb, w1, (((1,), (1,)), ((), ())),
        preferred_element_type=jnp.float32,
        precision=jax.lax.Precision.HIGHEST)
    gu = g * jax.nn.sigmoid(g) * u
    eo = jax.lax.dot_general(
        gu, w2, (((1,), (1,)), ((), ())),
        preferred_element_type=jnp.float32,
        precision=jax.lax.Precision.HIGHEST)

    acc = tw[:, None] * eo

    @pl.when(e == 0)
    def _():
        out_ref[...] = acc

    @pl.when(e > 0)
    def _():
        out_ref[...] += acc


def kernel(x, Wg, W_gate, W1, W2):
    grid = (N_TOKENS // TB, NUM_EXPERTS)
    return pl.pallas_call(
        _moe_body,
        grid=grid,
        in_specs=[
            pl.BlockSpec((TB, DIM), lambda t, e: (t, 0)),
            pl.BlockSpec((NUM_EXPERTS, DIM), lambda t, e: (0, 0)),
            pl.BlockSpec((1, ADJ_HIDDEN, DIM), lambda t, e: (e, 0, 0)),
            pl.BlockSpec((1, ADJ_HIDDEN, DIM), lambda t, e: (e, 0, 0)),
            pl.BlockSpec((1, DIM, ADJ_HIDDEN), lambda t, e: (e, 0, 0)),
        ],
        out_specs=pl.BlockSpec((TB, DIM), lambda t, e: (t, 0)),
        out_shape=jax.ShapeDtypeStruct((N_TOKENS, DIM), jnp.float32),
        compiler_params=pltpu.CompilerParams(
            dimension_semantics=("parallel", "arbitrary"),
        ),
    )(x, Wg, W_gate, W1, W2)


# speedup vs baseline: 2.7506x; 2.7506x over previous
import jax
import jax.numpy as jnp
from jax.experimental import pallas as pl
from jax.experimental.pallas import tpu as pltpu

DIM = 1024
NUM_EXPERTS = 4
TOP_K = 2
ADJ_HIDDEN = 1368
N_TOKENS = 2048
TB = 256


def _moe_body(x_ref, wg_ref, wgate_ref, w1_ref, w2_ref, out_ref):
    e = pl.program_id(1)
    xb = x_ref[...]

    logits = jax.lax.dot_general(
        xb, wg_ref[...], (((1,), (1,)), ((), ())),
        preferred_element_type=jnp.float32,
        precision=jax.lax.Precision.DEFAULT)
    m = jnp.max(logits, axis=-1, keepdims=True)
    ex = jnp.exp(logits - m)
    probs = ex / jnp.sum(ex, axis=-1, keepdims=True)

    cols = [probs[:, c] for c in range(NUM_EXPERTS)]
    sel = []
    for c in range(NUM_EXPERTS):
        rank = jnp.zeros_like(cols[c])
        for j in range(NUM_EXPERTS):
            if j == c:
                continue
            beats = (cols[j] > cols[c]) | ((cols[j] == cols[c]) & (j < c))
            rank = rank + beats.astype(jnp.float32)
        sel.append((rank < TOP_K).astype(jnp.float32))
    sum_sel = sum(s * p for s, p in zip(sel, cols))
    tw = jnp.zeros_like(cols[0])
    for c in range(NUM_EXPERTS):
        tw = jnp.where(e == c, sel[c] * cols[c] / (sum_sel + 1e-8), tw)

    wgate = wgate_ref[0]
    w1 = w1_ref[0]
    w2 = w2_ref[0]
    g = jax.lax.dot_general(
        xb, wgate, (((1,), (1,)), ((), ())),
        preferred_element_type=jnp.float32,
        precision=jax.lax.Precision.DEFAULT)
    u = jax.lax.dot_general(
        xb, w1, (((1,), (1,)), ((), ())),
        preferred_element_type=jnp.float32,
        precision=jax.lax.Precision.DEFAULT)
    gu = g * jax.nn.sigmoid(g) * u
    eo = jax.lax.dot_general(
        gu, w2, (((1,), (1,)), ((), ())),
        preferred_element_type=jnp.float32,
        precision=jax.lax.Precision.DEFAULT)

    acc = tw[:, None] * eo

    @pl.when(e == 0)
    def _():
        out_ref[...] = acc

    @pl.when(e > 0)
    def _():
        out_ref[...] += acc


def kernel(x, Wg, W_gate, W1, W2):
    grid = (N_TOKENS // TB, NUM_EXPERTS)
    return pl.pallas_call(
        _moe_body,
        grid=grid,
        in_specs=[
            pl.BlockSpec((TB, DIM), lambda t, e: (t, 0)),
            pl.BlockSpec((NUM_EXPERTS, DIM), lambda t, e: (0, 0)),
            pl.BlockSpec((1, ADJ_HIDDEN, DIM), lambda t, e: (e, 0, 0)),
            pl.BlockSpec((1, ADJ_HIDDEN, DIM), lambda t, e: (e, 0, 0)),
            pl.BlockSpec((1, DIM, ADJ_HIDDEN), lambda t, e: (e, 0, 0)),
        ],
        out_specs=pl.BlockSpec((TB, DIM), lambda t, e: (t, 0)),
        out_shape=jax.ShapeDtypeStruct((N_TOKENS, DIM), jnp.float32),
        compiler_params=pltpu.CompilerParams(
            dimension_semantics=("parallel", "arbitrary"),
        ),
    )(x, Wg, W_gate, W1, W2)


# speedup vs baseline: 3.6769x; 1.3368x over previous
import jax
import jax.numpy as jnp
from jax.experimental import pallas as pl
from jax.experimental.pallas import tpu as pltpu

DIM = 1024
NUM_EXPERTS = 4
TOP_K = 2
ADJ_HIDDEN = 1368
N_TOKENS = 2048
TB = 256


def _router_tw(xb, wg):
    logits = jax.lax.dot_general(
        xb, wg, (((1,), (1,)), ((), ())),
        preferred_element_type=jnp.float32,
        precision=jax.lax.Precision.DEFAULT)
    m = jnp.max(logits, axis=-1, keepdims=True)
    ex = jnp.exp(logits - m)
    probs = ex / jnp.sum(ex, axis=-1, keepdims=True)
    cols = [probs[:, c] for c in range(NUM_EXPERTS)]
    sel = []
    for c in range(NUM_EXPERTS):
        rank = jnp.zeros_like(cols[c])
        for j in range(NUM_EXPERTS):
            if j == c:
                continue
            beats = (cols[j] > cols[c]) | ((cols[j] == cols[c]) & (j < c))
            rank = rank + beats.astype(jnp.float32)
        sel.append((rank < TOP_K).astype(jnp.float32))
    sum_sel = sum(s * p for s, p in zip(sel, cols))
    tw = [s * p / (sum_sel + 1e-8) for s, p in zip(sel, cols)]
    return jnp.stack(tw, axis=-1)


def _moe_body(x_ref, wg_ref, wgate_ref, w1_ref, w2_ref, out_ref, tw_ref):
    e = pl.program_id(0)
    tb = pl.program_id(1)
    xb = x_ref[...]

    @pl.when(e == 0)
    def _():
        tw_ref[pl.ds(tb * TB, TB), :] = _router_tw(xb, wg_ref[...])
    wgate = wgate_ref[0]
    w1 = w1_ref[0]
    w2 = w2_ref[0]
    g = jax.lax.dot_general(
        xb, wgate, (((1,), (1,)), ((), ())),
        preferred_element_type=jnp.float32,
        precision=jax.lax.Precision.DEFAULT)
    u = jax.lax.dot_general(
        xb, w1, (((1,), (1,)), ((), ())),
        preferred_element_type=jnp.float32,
        precision=jax.lax.Precision.DEFAULT)
    gu = g * jax.nn.sigmoid(g) * u
    eo = jax.lax.dot_general(
        gu, w2, (((1,), (1,)), ((), ())),
        preferred_element_type=jnp.float32,
        precision=jax.lax.Precision.DEFAULT)

    tw_blk = tw_ref[pl.ds(tb * TB, TB), :]
    lane = jax.lax.broadcasted_iota(jnp.int32, (TB, NUM_EXPERTS), 1)
    tw_e = jnp.sum(tw_blk * (lane == e).astype(jnp.float32), axis=1)

    acc = tw_e[:, None] * eo

    @pl.when(e == 0)
    def _():
        out_ref[pl.ds(tb * TB, TB), :] = acc

    @pl.when(e > 0)
    def _():
        out_ref[pl.ds(tb * TB, TB), :] += acc


def kernel(x, Wg, W_gate, W1, W2):
    grid = (NUM_EXPERTS, N_TOKENS // TB)
    return pl.pallas_call(
        _moe_body,
        grid=grid,
        in_specs=[
            pl.BlockSpec((TB, DIM), lambda e, t: (t, 0)),
            pl.BlockSpec((NUM_EXPERTS, DIM), lambda e, t: (0, 0)),
            pl.BlockSpec((1, ADJ_HIDDEN, DIM), lambda e, t: (e, 0, 0)),
            pl.BlockSpec((1, ADJ_HIDDEN, DIM), lambda e, t: (e, 0, 0)),
            pl.BlockSpec((1, DIM, ADJ_HIDDEN), lambda e, t: (e, 0, 0)),
        ],
        out_specs=pl.BlockSpec((N_TOKENS, DIM), lambda e, t: (0, 0)),
        out_shape=jax.ShapeDtypeStruct((N_TOKENS, DIM), jnp.float32),
        scratch_shapes=[pltpu.VMEM((N_TOKENS, NUM_EXPERTS), jnp.float32)],
        compiler_params=pltpu.CompilerParams(
            dimension_semantics=("arbitrary", "arbitrary"),
        ),
    )(x, Wg, W_gate, W1, W2)
